# baseline (speedup 1.0000x reference)
import functools

import jax
import jax.numpy as jnp
from jax import lax
from jax.experimental import pallas as pl
from jax.experimental.pallas import tpu as pltpu
from jax.experimental.pallas import tpu_sc as plsc

N = 1 << 21
RES = 128
C = 128
R = N // C
BR = 512
NBLK = R // BR

_KS0 = 0
_KS1 = 42
_KS2 = _KS0 ^ _KS1 ^ 0x1BD11BDA
_ROTS = ((13, 15, 26, 6), (17, 29, 16, 24))


def _rotl(v, r):
    return (v << jnp.uint32(r)) | (v >> jnp.uint32(32 - r))


def _threefry_bits(g):
    ks = (jnp.uint32(_KS0), jnp.uint32(_KS1), jnp.uint32(_KS2))
    x0 = jnp.zeros_like(g) + ks[0]
    x1 = g + ks[1]
    for r in range(5):
        for d in _ROTS[r % 2]:
            x0 = x0 + x1
            x1 = _rotl(x1, d)
            x1 = x0 ^ x1
        x0 = x0 + ks[(r + 1) % 3]
        x1 = x1 + ks[(r + 2) % 3] + jnp.uint32(r + 1)
    return x0 ^ x1


def _tcb_body(thr_ref):
    i = pl.program_id(0)
    row = lax.broadcasted_iota(jnp.uint32, (BR, C), 0)
    col = lax.broadcasted_iota(jnp.uint32, (BR, C), 1)
    g = (jnp.uint32(BR) * i.astype(jnp.uint32) + row) * jnp.uint32(C) + col
    bits = _threefry_bits(g)
    fb = (bits >> jnp.uint32(9)) | jnp.uint32(0x3F800000)
    u = lax.bitcast_convert_type(fb, jnp.float32) - 1.0
    t_u = -jnp.log1p(-u) - 1e-4
    thr_ref[...] = jnp.minimum(jnp.float32(0.01), t_u)


_tcb = pl.pallas_call(
    _tcb_body,
    grid=(NBLK,),
    out_specs=pl.BlockSpec((BR, C), lambda i: (i, 0)),
    out_shape=jax.ShapeDtypeStruct((R, C), jnp.float32),
)


def _tcc_body(d_ref, thr_ref, o_ref):
    o_ref[...] = (d_ref[...] > thr_ref[...]).astype(jnp.uint8)


_tcc = pl.pallas_call(
    _tcc_body,
    grid=(NBLK,),
    in_specs=[pl.BlockSpec((BR, C), lambda i: (i, 0)),
              pl.BlockSpec((BR, C), lambda i: (i, 0))],
    out_specs=pl.BlockSpec((BR, C), lambda i: (i, 0)),
    out_shape=jax.ShapeDtypeStruct((R, C), jnp.uint8),
)

_NC = 2
_NS = 16
_NW = _NC * _NS
_NP = N // _NW
_CH = 8192
_CROWS = _CH // C
_NCH = _NP // _CH
_RM = 12582912.0


def _voxf(v):
    f = (v + 1.0) * 64.0 - 0.5
    f = (f + _RM) - _RM
    return jnp.minimum(jnp.maximum(f, 0.0), 127.0)


def _sc_compute_chunk(xv, yv, zv, idx_v, pen_v):
    def body(i, _):
        p = i << 4
        x = xv[pl.ds(p, 16)]
        y = yv[pl.ds(p, 16)]
        z = zv[pl.ds(p, 16)]
        linf = (_voxf(z) * 128.0 + _voxf(y)) * 128.0 + _voxf(x)
        idx_v[pl.ds(p, 16)] = linf.astype(jnp.int32)
        m = jnp.maximum(jnp.maximum(jnp.abs(x), jnp.abs(y)), jnp.abs(z))
        pen_v[pl.ds(p, 16)] = jnp.where(m <= 1.0, jnp.zeros_like(x),
                                        jnp.full_like(x, -2.0))
        return 0

    lax.fori_loop(0, _CH // 16, body, 0)


def _sc_pass2(d_v, pen_v):
    def body(i, _):
        p = i << 4
        d_v[pl.ds(p, 16)] = d_v[pl.ds(p, 16)] + pen_v[pl.ds(p, 16)]
        return 0

    lax.fori_loop(0, _CH // 16, body, 0)


@functools.cache
def _make_sc_gather():
    mesh = plsc.VectorSubcoreMesh(core_axis_name="c", subcore_axis_name="s")

    slab = pltpu.VMEM((_CH,), jnp.float32)
    flat_i = pltpu.VMEM((_CH,), jnp.int32)
    flat_f = pltpu.VMEM((_CH,), jnp.float32)

    @functools.partial(
        pl.kernel,
        mesh=mesh,
        out_type=jax.ShapeDtypeStruct((N,), jnp.float32),
        scratch_types=[slab, slab, slab, slab, slab, slab,
                       flat_i, flat_i, flat_f, flat_f, flat_f, flat_f,
                       pltpu.SemaphoreType.DMA, pltpu.SemaphoreType.DMA],
    )
    def _sc_gather(xt_hbm, grid_hbm, d_hbm,
                   xv0, yv0, zv0, xv1, yv1, zv1,
                   idx0, idx1, pen0, pen1, dv0, dv1, s0, s1):
        wid = lax.axis_index("s") * _NC + lax.axis_index("c")
        base0 = wid * _NP
        xvs, yvs, zvs = (xv0, xv1), (yv0, yv1), (zv0, zv1)
        idxs, pens, dvs, sems = (idx0, idx1), (pen0, pen1), (dv0, dv1), (s0, s1)

        def load_and_compute(k, b):
            p0 = base0 + k * _CH
            pltpu.sync_copy(xt_hbm.at[pl.ds(p0, _CH)], xvs[b])
            pltpu.sync_copy(xt_hbm.at[pl.ds(N + p0, _CH)], yvs[b])
            pltpu.sync_copy(xt_hbm.at[pl.ds(2 * N + p0, _CH)], zvs[b])
            _sc_compute_chunk(xvs[b], yvs[b], zvs[b], idxs[b], pens[b])

        def drain(k):
            b = k % 2
            _sc_pass2(dvs[b], pens[b])
            pltpu.sync_copy(dvs[b], d_hbm.at[pl.ds(base0 + k * _CH, _CH)])

        load_and_compute(0, 0)
        cps = [pltpu.async_copy(grid_hbm.at[idx0], dv0, s0)]
        for k in range(1, _NCH):
            b = k % 2
            load_and_compute(k, b)
            cps.append(pltpu.async_copy(grid_hbm.at[idxs[b]], dvs[b], sems[b]))
            cps[k - 1].wait()
            drain(k - 1)
        cps[_NCH - 1].wait()
        drain(_NCH - 1)

    return _sc_gather


def kernel(xyz_ndc, grid):
    xt = jnp.transpose(xyz_ndc).reshape(3 * N)
    d = _make_sc_gather()(xt, grid.reshape(N))
    thr = _tcb()
    out8 = _tcc(d.reshape(R, C), thr)
    return out8.reshape(N).view(jnp.bool_)


# speedup vs baseline: 3.4199x; 3.4199x over previous
import functools

import jax
import jax.numpy as jnp
from jax import lax
from jax.experimental import pallas as pl
from jax.experimental.pallas import tpu as pltpu
from jax.experimental.pallas import tpu_sc as plsc

N = 1 << 21
RES = 128
C = 128
R = N // C
BR = 512
NBLK = R // BR

_KS0 = 0
_KS1 = 42
_KS2 = _KS0 ^ _KS1 ^ 0x1BD11BDA
_ROTS = ((13, 15, 26, 6), (17, 29, 16, 24))


def _rotl(v, r):
    return (v << jnp.uint32(r)) | (v >> jnp.uint32(32 - r))


def _threefry_bits(g):
    ks = (jnp.uint32(_KS0), jnp.uint32(_KS1), jnp.uint32(_KS2))
    x0 = jnp.zeros_like(g) + ks[0]
    x1 = g + ks[1]
    for r in range(5):
        for d in _ROTS[r % 2]:
            x0 = x0 + x1
            x1 = _rotl(x1, d)
            x1 = x0 ^ x1
        x0 = x0 + ks[(r + 1) % 3]
        x1 = x1 + ks[(r + 2) % 3] + jnp.uint32(r + 1)
    return x0 ^ x1


def _tcb_body(thr_ref):
    i = pl.program_id(0)
    row = lax.broadcasted_iota(jnp.uint32, (BR, C), 0)
    col = lax.broadcasted_iota(jnp.uint32, (BR, C), 1)
    g = (jnp.uint32(BR) * i.astype(jnp.uint32) + row) * jnp.uint32(C) + col
    bits = _threefry_bits(g)
    fb = (bits >> jnp.uint32(9)) | jnp.uint32(0x3F800000)
    u = lax.bitcast_convert_type(fb, jnp.float32) - 1.0
    t_u = -jnp.log1p(-u) - 1e-4
    thr_ref[...] = jnp.minimum(jnp.float32(0.01), t_u)


_tcb = pl.pallas_call(
    _tcb_body,
    grid=(NBLK,),
    out_specs=pl.BlockSpec((BR, C), lambda i: (i, 0)),
    out_shape=jax.ShapeDtypeStruct((R, C), jnp.float32),
)


def _tcc_body(d_ref, thr_ref, o_ref):
    o_ref[...] = (d_ref[...] > thr_ref[...]).astype(jnp.uint8)


_tcc = pl.pallas_call(
    _tcc_body,
    grid=(NBLK,),
    in_specs=[pl.BlockSpec((BR, C), lambda i: (i, 0)),
              pl.BlockSpec((BR, C), lambda i: (i, 0))],
    out_specs=pl.BlockSpec((BR, C), lambda i: (i, 0)),
    out_shape=jax.ShapeDtypeStruct((R, C), jnp.uint8),
)

_NC = 2
_NS = 16
_NW = _NC * _NS
_NP = N // _NW
_CH = 8192
_CROWS = _CH // C
_NCH = _NP // _CH
_RM = 12582912.0


def _voxf(v):
    f = (v + 1.0) * 64.0 - 0.5
    f = (f + _RM) - _RM
    return jnp.minimum(jnp.maximum(f, 0.0), 127.0)


def _sc_compute_chunk(xv, yv, zv, idx_v, pen_v):
    def body(i, _):
        row = i >> 3
        co = (i & 7) << 4
        p = i << 4
        x = xv[row, pl.ds(co, 16)]
        y = yv[row, pl.ds(co, 16)]
        z = zv[row, pl.ds(co, 16)]
        linf = (_voxf(z) * 128.0 + _voxf(y)) * 128.0 + _voxf(x)
        idx_v[pl.ds(p, 16)] = linf.astype(jnp.int32)
        m = jnp.maximum(jnp.maximum(jnp.abs(x), jnp.abs(y)), jnp.abs(z))
        pen_v[pl.ds(p, 16)] = jnp.where(m <= 1.0, jnp.zeros_like(x),
                                        jnp.full_like(x, -2.0))
        return 0

    lax.fori_loop(0, _CH // 16, body, 0)


def _sc_pass2(d_v, pen_v):
    def body(i, _):
        p = i << 4
        d_v[pl.ds(p, 16)] = d_v[pl.ds(p, 16)] + pen_v[pl.ds(p, 16)]
        return 0

    lax.fori_loop(0, _CH // 16, body, 0)


@functools.cache
def _make_sc_gather():
    mesh = plsc.VectorSubcoreMesh(core_axis_name="c", subcore_axis_name="s")

    slab = pltpu.VMEM((_CROWS, C), jnp.float32)
    flat_i = pltpu.VMEM((_CH,), jnp.int32)
    flat_f = pltpu.VMEM((_CH,), jnp.float32)

    @functools.partial(
        pl.kernel,
        mesh=mesh,
        out_type=jax.ShapeDtypeStruct((N,), jnp.float32),
        scratch_types=[slab, slab, slab, slab, slab, slab,
                       flat_i, flat_i, flat_f, flat_f, flat_f, flat_f,
                       pltpu.SemaphoreType.DMA, pltpu.SemaphoreType.DMA],
    )
    def _sc_gather(xt_hbm, grid_hbm, d_hbm,
                   xv0, yv0, zv0, xv1, yv1, zv1,
                   idx0, idx1, pen0, pen1, dv0, dv1, s0, s1):
        wid = lax.axis_index("s") * _NC + lax.axis_index("c")
        base0 = wid * _NP
        xvs, yvs, zvs = (xv0, xv1), (yv0, yv1), (zv0, zv1)
        idxs, pens, dvs, sems = (idx0, idx1), (pen0, pen1), (dv0, dv1), (s0, s1)

        row0 = wid * (_NP // C)

        def load_and_compute(k, b):
            r = row0 + k * _CROWS
            pltpu.sync_copy(xt_hbm.at[0, pl.ds(r, _CROWS), :], xvs[b])
            pltpu.sync_copy(xt_hbm.at[1, pl.ds(r, _CROWS), :], yvs[b])
            pltpu.sync_copy(xt_hbm.at[2, pl.ds(r, _CROWS), :], zvs[b])
            _sc_compute_chunk(xvs[b], yvs[b], zvs[b], idxs[b], pens[b])

        def drain(k):
            b = k % 2
            _sc_pass2(dvs[b], pens[b])
            pltpu.sync_copy(dvs[b], d_hbm.at[pl.ds(base0 + k * _CH, _CH)])

        load_and_compute(0, 0)
        cps = [pltpu.async_copy(grid_hbm.at[idx0], dv0, s0)]
        for k in range(1, _NCH):
            b = k % 2
            load_and_compute(k, b)
            cps.append(pltpu.async_copy(grid_hbm.at[idxs[b]], dvs[b], sems[b]))
            cps[k - 1].wait()
            drain(k - 1)
        cps[_NCH - 1].wait()
        drain(_NCH - 1)

    return _sc_gather


def kernel(xyz_ndc, grid):
    xt = jnp.transpose(xyz_ndc).reshape(3, R, C)
    d = _make_sc_gather()(xt, grid.reshape(N))
    thr = _tcb()
    out8 = _tcc(d.reshape(R, C), thr)
    return out8.reshape(N).view(jnp.bool_)


# speedup vs baseline: 3.4346x; 1.0043x over previous
import functools

import jax
import jax.numpy as jnp
from jax import lax
from jax.experimental import pallas as pl
from jax.experimental.pallas import tpu as pltpu
from jax.experimental.pallas import tpu_sc as plsc

N = 1 << 21
RES = 128
C = 128
R = N // C
BR = 512
NBLK = R // BR

_KS0 = 0
_KS1 = 42
_KS2 = _KS0 ^ _KS1 ^ 0x1BD11BDA
_ROTS = ((13, 15, 26, 6), (17, 29, 16, 24))


def _rotl(v, r):
    return (v << jnp.uint32(r)) | (v >> jnp.uint32(32 - r))


def _threefry_bits(g):
    ks = (jnp.uint32(_KS0), jnp.uint32(_KS1), jnp.uint32(_KS2))
    x0 = jnp.zeros_like(g) + ks[0]
    x1 = g + ks[1]
    for r in range(5):
        for d in _ROTS[r % 2]:
            x0 = x0 + x1
            x1 = _rotl(x1, d)
            x1 = x0 ^ x1
        x0 = x0 + ks[(r + 1) % 3]
        x1 = x1 + ks[(r + 2) % 3] + jnp.uint32(r + 1)
    return x0 ^ x1


def _tcb_body(thr_ref):
    i = pl.program_id(0)
    row = lax.broadcasted_iota(jnp.uint32, (BR, C), 0)
    col = lax.broadcasted_iota(jnp.uint32, (BR, C), 1)
    g = (jnp.uint32(BR) * i.astype(jnp.uint32) + row) * jnp.uint32(C) + col
    bits = _threefry_bits(g)
    fb = (bits >> jnp.uint32(9)) | jnp.uint32(0x3F800000)
    u = lax.bitcast_convert_type(fb, jnp.float32) - 1.0
    t_u = -jnp.log1p(-u) - 1e-4
    thr_ref[...] = jnp.minimum(jnp.float32(0.01), t_u)


_tcb = pl.pallas_call(
    _tcb_body,
    grid=(NBLK,),
    out_specs=pl.BlockSpec((BR, C), lambda i: (i, 0)),
    out_shape=jax.ShapeDtypeStruct((R, C), jnp.float32),
)


def _tcc_body(d_ref, thr_ref, o_ref):
    o_ref[...] = (d_ref[...] > thr_ref[...]).astype(jnp.uint8)


_tcc = pl.pallas_call(
    _tcc_body,
    grid=(NBLK,),
    in_specs=[pl.BlockSpec((BR, C), lambda i: (i, 0)),
              pl.BlockSpec((BR, C), lambda i: (i, 0))],
    out_specs=pl.BlockSpec((BR, C), lambda i: (i, 0)),
    out_shape=jax.ShapeDtypeStruct((R, C), jnp.uint8),
)

_NC = 2
_NS = 16
_NW = _NC * _NS
_NP = N // _NW
_CH = 8192
_CROWS = _CH // C
_NCH = _NP // _CH
_RM = 12582912.0


def _voxf(v):
    f = (v + 1.0) * 64.0 - 0.5
    f = (f + _RM) - _RM
    return jnp.minimum(jnp.maximum(f, 0.0), 127.0)


def _sc_compute_chunk(xv, yv, zv, idx_v, pen_v):
    @plsc.parallel_loop(0, _CH, step=16, unroll=4)
    def _(p):
        row = p >> 7
        co = p & 127
        x = xv[row, pl.ds(co, 16)]
        y = yv[row, pl.ds(co, 16)]
        z = zv[row, pl.ds(co, 16)]
        linf = (_voxf(z) * 128.0 + _voxf(y)) * 128.0 + _voxf(x)
        idx_v[pl.ds(p, 16)] = linf.astype(jnp.int32)
        m = jnp.maximum(jnp.maximum(jnp.abs(x), jnp.abs(y)), jnp.abs(z))
        pen_v[pl.ds(p, 16)] = jnp.where(m <= 1.0, jnp.zeros_like(x),
                                        jnp.full_like(x, -2.0))


def _sc_pass2(d_v, pen_v):
    @plsc.parallel_loop(0, _CH, step=16, unroll=4)
    def _(p):
        d_v[pl.ds(p, 16)] = d_v[pl.ds(p, 16)] + pen_v[pl.ds(p, 16)]


@functools.cache
def _make_sc_gather():
    mesh = plsc.VectorSubcoreMesh(core_axis_name="c", subcore_axis_name="s")

    slab = pltpu.VMEM((_CROWS, C), jnp.float32)
    flat_i = pltpu.VMEM((_CH,), jnp.int32)
    flat_f = pltpu.VMEM((_CH,), jnp.float32)

    @functools.partial(
        pl.kernel,
        mesh=mesh,
        out_type=jax.ShapeDtypeStruct((N,), jnp.float32),
        scratch_types=[slab, slab, slab, slab, slab, slab,
                       flat_i, flat_i, flat_f, flat_f, flat_f, flat_f,
                       pltpu.SemaphoreType.DMA, pltpu.SemaphoreType.DMA],
    )
    def _sc_gather(xt_hbm, grid_hbm, d_hbm,
                   xv0, yv0, zv0, xv1, yv1, zv1,
                   idx0, idx1, pen0, pen1, dv0, dv1, s0, s1):
        wid = lax.axis_index("s") * _NC + lax.axis_index("c")
        base0 = wid * _NP
        xvs, yvs, zvs = (xv0, xv1), (yv0, yv1), (zv0, zv1)
        idxs, pens, dvs, sems = (idx0, idx1), (pen0, pen1), (dv0, dv1), (s0, s1)

        row0 = wid * (_NP // C)

        def load_and_compute(k, b):
            r = row0 + k * _CROWS
            pltpu.sync_copy(xt_hbm.at[0, pl.ds(r, _CROWS), :], xvs[b])
            pltpu.sync_copy(xt_hbm.at[1, pl.ds(r, _CROWS), :], yvs[b])
            pltpu.sync_copy(xt_hbm.at[2, pl.ds(r, _CROWS), :], zvs[b])
            _sc_compute_chunk(xvs[b], yvs[b], zvs[b], idxs[b], pens[b])

        def drain(k):
            b = k % 2
            _sc_pass2(dvs[b], pens[b])
            pltpu.sync_copy(dvs[b], d_hbm.at[pl.ds(base0 + k * _CH, _CH)])

        load_and_compute(0, 0)
        cps = [pltpu.async_copy(grid_hbm.at[idx0], dv0, s0)]
        for k in range(1, _NCH):
            b = k % 2
            load_and_compute(k, b)
            cps.append(pltpu.async_copy(grid_hbm.at[idxs[b]], dvs[b], sems[b]))
            cps[k - 1].wait()
            drain(k - 1)
        cps[_NCH - 1].wait()
        drain(_NCH - 1)

    return _sc_gather


def kernel(xyz_ndc, grid):
    xt = jnp.transpose(xyz_ndc).reshape(3, R, C)
    d = _make_sc_gather()(xt, grid.reshape(N))
    thr = _tcb()
    out8 = _tcc(d.reshape(R, C), thr)
    return out8.reshape(N).view(jnp.bool_)


# speedup vs baseline: 3.5656x; 1.0382x over previous
import functools

import jax
import jax.numpy as jnp
from jax import lax
from jax.experimental import pallas as pl
from jax.experimental.pallas import tpu as pltpu
from jax.experimental.pallas import tpu_sc as plsc

N = 1 << 21
RES = 128
C = 128
R = N // C
BR = 512
NBLK = R // BR

_KS0 = 0
_KS1 = 42
_KS2 = _KS0 ^ _KS1 ^ 0x1BD11BDA
_ROTS = ((13, 15, 26, 6), (17, 29, 16, 24))


def _rotl(v, r):
    return (v << jnp.uint32(r)) | (v >> jnp.uint32(32 - r))


def _threefry_bits(g):
    ks = (jnp.uint32(_KS0), jnp.uint32(_KS1), jnp.uint32(_KS2))
    x0 = jnp.zeros_like(g) + ks[0]
    x1 = g + ks[1]
    for r in range(5):
        for d in _ROTS[r % 2]:
            x0 = x0 + x1
            x1 = _rotl(x1, d)
            x1 = x0 ^ x1
        x0 = x0 + ks[(r + 1) % 3]
        x1 = x1 + ks[(r + 2) % 3] + jnp.uint32(r + 1)
    return x0 ^ x1


def _tcb_body(thr_ref):
    i = pl.program_id(0)
    row = lax.broadcasted_iota(jnp.uint32, (BR, C), 0)
    col = lax.broadcasted_iota(jnp.uint32, (BR, C), 1)
    g = (jnp.uint32(BR) * i.astype(jnp.uint32) + row) * jnp.uint32(C) + col
    bits = _threefry_bits(g)
    fb = (bits >> jnp.uint32(9)) | jnp.uint32(0x3F800000)
    u = lax.bitcast_convert_type(fb, jnp.float32) - 1.0
    t_u = -jnp.log1p(-u) - 1e-4
    thr_ref[...] = jnp.minimum(jnp.float32(0.01), t_u)


_tcb = pl.pallas_call(
    _tcb_body,
    grid=(NBLK,),
    out_specs=pl.BlockSpec((BR, C), lambda i: (i, 0)),
    out_shape=jax.ShapeDtypeStruct((R, C), jnp.float32),
)


def _tcc_body(d_ref, thr_ref, o_ref):
    o_ref[...] = (d_ref[...] > thr_ref[...]).astype(jnp.uint8)


_tcc = pl.pallas_call(
    _tcc_body,
    grid=(NBLK,),
    in_specs=[pl.BlockSpec((BR, C), lambda i: (i, 0)),
              pl.BlockSpec((BR, C), lambda i: (i, 0))],
    out_specs=pl.BlockSpec((BR, C), lambda i: (i, 0)),
    out_shape=jax.ShapeDtypeStruct((R, C), jnp.uint8),
)

_NC = 2
_NS = 16
_NW = _NC * _NS
_NP = N // _NW
_CH = 8192
_CROWS = _CH // C
_NCH = _NP // _CH
_RM = 12582912.0


def _voxf(v):
    f = (v + 1.0) * 64.0 - 0.5
    f = (f + _RM) - _RM
    return jnp.minimum(jnp.maximum(f, 0.0), 127.0)


def _sc_compute_chunk(sv, idx_v, pen_v):
    @plsc.parallel_loop(0, _CH, step=16, unroll=4)
    def _(p):
        row = p >> 7
        co = p & 127
        x = sv[0, row, pl.ds(co, 16)]
        y = sv[1, row, pl.ds(co, 16)]
        z = sv[2, row, pl.ds(co, 16)]
        linf = (_voxf(z) * 128.0 + _voxf(y)) * 128.0 + _voxf(x)
        idx_v[pl.ds(p, 16)] = linf.astype(jnp.int32)
        m = jnp.maximum(jnp.maximum(jnp.abs(x), jnp.abs(y)), jnp.abs(z))
        pen_v[pl.ds(p, 16)] = jnp.where(m <= 1.0, jnp.zeros_like(x),
                                        jnp.full_like(x, -2.0))


def _sc_pass2(d_v, pen_v):
    @plsc.parallel_loop(0, _CH, step=16, unroll=4)
    def _(p):
        d_v[pl.ds(p, 16)] = d_v[pl.ds(p, 16)] + pen_v[pl.ds(p, 16)]


@functools.cache
def _make_sc_gather():
    mesh = plsc.VectorSubcoreMesh(core_axis_name="c", subcore_axis_name="s")

    slab3 = pltpu.VMEM((3, _CROWS, C), jnp.float32)
    flat_i = pltpu.VMEM((_CH,), jnp.int32)
    flat_f = pltpu.VMEM((_CH,), jnp.float32)

    @functools.partial(
        pl.kernel,
        mesh=mesh,
        out_type=jax.ShapeDtypeStruct((N,), jnp.float32),
        scratch_types=[slab3, slab3,
                       flat_i, flat_i, flat_f, flat_f, flat_f, flat_f,
                       pltpu.SemaphoreType.DMA, pltpu.SemaphoreType.DMA,
                       pltpu.SemaphoreType.DMA, pltpu.SemaphoreType.DMA],
    )
    def _sc_gather(xt_hbm, grid_hbm, d_hbm,
                   sv0, sv1, idx0, idx1, pen0, pen1, dv0, dv1,
                   s0, s1, o0, o1):
        wid = lax.axis_index("s") * _NC + lax.axis_index("c")
        base0 = wid * _NP
        svs = (sv0, sv1)
        idxs, pens, dvs = (idx0, idx1), (pen0, pen1), (dv0, dv1)
        sems, osems = (s0, s1), (o0, o1)

        row0 = wid * (_NP // C)

        def load_and_compute(k, b):
            r = row0 + k * _CROWS
            pltpu.sync_copy(xt_hbm.at[:, pl.ds(r, _CROWS), :], svs[b])
            _sc_compute_chunk(svs[b], idxs[b], pens[b])

        load_and_compute(0, 0)
        cps = [pltpu.async_copy(grid_hbm.at[idx0], dv0, s0)]
        wrs = [None] * _NCH
        for k in range(1, _NCH):
            b = k % 2
            load_and_compute(k, b)
            if k >= 2:
                wrs[k - 2].wait()
            cps.append(pltpu.async_copy(grid_hbm.at[idxs[b]], dvs[b], sems[b]))
            cps[k - 1].wait()
            _sc_pass2(dvs[1 - b], pens[1 - b])
            wrs[k - 1] = pltpu.async_copy(
                dvs[1 - b], d_hbm.at[pl.ds(base0 + (k - 1) * _CH, _CH)],
                osems[1 - b])
        last = _NCH - 1
        cps[last].wait()
        _sc_pass2(dvs[last % 2], pens[last % 2])
        wrs[last - 1].wait()
        wrs[last] = pltpu.async_copy(
            dvs[last % 2], d_hbm.at[pl.ds(base0 + last * _CH, _CH)],
            osems[last % 2])
        wrs[last].wait()

    return _sc_gather


def kernel(xyz_ndc, grid):
    xt = jnp.transpose(xyz_ndc).reshape(3, R, C)
    d = _make_sc_gather()(xt, grid.reshape(N))
    thr = _tcb()
    out8 = _tcc(d.reshape(R, C), thr)
    return out8.reshape(N).view(jnp.bool_)


# speedup vs baseline: 3.6678x; 1.0287x over previous
import functools

import jax
import jax.numpy as jnp
from jax import lax
from jax.experimental import pallas as pl
from jax.experimental.pallas import tpu as pltpu
from jax.experimental.pallas import tpu_sc as plsc

N = 1 << 21
RES = 128
C = 128
R = N // C
BR = 512
NBLK = R // BR

_KS0 = 0
_KS1 = 42
_KS2 = _KS0 ^ _KS1 ^ 0x1BD11BDA
_ROTS = ((13, 15, 26, 6), (17, 29, 16, 24))


def _rotl(v, r):
    return (v << jnp.uint32(r)) | (v >> jnp.uint32(32 - r))


def _threefry_bits(g):
    ks = (jnp.uint32(_KS0), jnp.uint32(_KS1), jnp.uint32(_KS2))
    x0 = jnp.zeros_like(g) + ks[0]
    x1 = g + ks[1]
    for r in range(5):
        for d in _ROTS[r % 2]:
            x0 = x0 + x1
            x1 = _rotl(x1, d)
            x1 = x0 ^ x1
        x0 = x0 + ks[(r + 1) % 3]
        x1 = x1 + ks[(r + 2) % 3] + jnp.uint32(r + 1)
    return x0 ^ x1


def _tcb_body(thr_ref):
    i = pl.program_id(0)
    row = lax.broadcasted_iota(jnp.uint32, (BR, C), 0)
    col = lax.broadcasted_iota(jnp.uint32, (BR, C), 1)
    g = (jnp.uint32(BR) * i.astype(jnp.uint32) + row) * jnp.uint32(C) + col
    bits = _threefry_bits(g)
    fb = (bits >> jnp.uint32(9)) | jnp.uint32(0x3F800000)
    u = lax.bitcast_convert_type(fb, jnp.float32) - 1.0
    t_u = -jnp.log1p(-u) - 1e-4
    thr_ref[...] = jnp.minimum(jnp.float32(0.01), t_u)


_tcb = pl.pallas_call(
    _tcb_body,
    grid=(NBLK,),
    out_specs=pl.BlockSpec((BR, C), lambda i: (i, 0)),
    out_shape=jax.ShapeDtypeStruct((R, C), jnp.float32),
)


def _tcc_body(d_ref, thr_ref, o_ref):
    o_ref[...] = (d_ref[...] > thr_ref[...]).astype(jnp.uint8)


_tcc = pl.pallas_call(
    _tcc_body,
    grid=(NBLK,),
    in_specs=[pl.BlockSpec((BR, C), lambda i: (i, 0)),
              pl.BlockSpec((BR, C), lambda i: (i, 0))],
    out_specs=pl.BlockSpec((BR, C), lambda i: (i, 0)),
    out_shape=jax.ShapeDtypeStruct((R, C), jnp.uint8),
)

_NC = 2
_NS = 16
_NW = _NC * _NS
_NP = N // _NW
_CH = 8192
_CROWS = _CH // C
_NCH = _NP // _CH
_RM = 12582912.0


def _voxf(v):
    f = (v + 1.0) * 64.0 - 0.5
    f = (f + _RM) - _RM
    return jnp.minimum(jnp.maximum(f, 0.0), 127.0)


def _sc_compute_chunk(sv, idx_v, pen_v):
    @plsc.parallel_loop(0, _CH, step=16, unroll=4)
    def _(p):
        row = p >> 7
        co = p & 127
        x = sv[0, row, pl.ds(co, 16)]
        y = sv[1, row, pl.ds(co, 16)]
        z = sv[2, row, pl.ds(co, 16)]
        linf = (_voxf(z) * 128.0 + _voxf(y)) * 128.0 + _voxf(x)
        idx_v[pl.ds(p, 16)] = linf.astype(jnp.int32)
        m = jnp.maximum(jnp.maximum(jnp.abs(x), jnp.abs(y)), jnp.abs(z))
        pen_v[pl.ds(p, 16)] = jnp.where(m <= 1.0, jnp.zeros_like(x),
                                        jnp.full_like(x, -2.0))


def _sc_pass2(d_v, pen_v):
    @plsc.parallel_loop(0, _CH, step=16, unroll=4)
    def _(p):
        d_v[pl.ds(p, 16)] = d_v[pl.ds(p, 16)] + pen_v[pl.ds(p, 16)]


@functools.cache
def _make_sc_gather():
    mesh = plsc.VectorSubcoreMesh(core_axis_name="c", subcore_axis_name="s")

    slab3 = pltpu.VMEM((3, _CROWS, C), jnp.float32)
    flat_i = pltpu.VMEM((_CH,), jnp.int32)
    flat_f = pltpu.VMEM((_CH,), jnp.float32)

    @functools.partial(
        pl.kernel,
        mesh=mesh,
        out_type=jax.ShapeDtypeStruct((N,), jnp.float32),
        scratch_types=[slab3, slab3,
                       flat_i, flat_i, flat_f, flat_f, flat_f, flat_f,
                       pltpu.SemaphoreType.DMA, pltpu.SemaphoreType.DMA,
                       pltpu.SemaphoreType.DMA, pltpu.SemaphoreType.DMA,
                       pltpu.SemaphoreType.DMA],
    )
    def _sc_gather(xt_hbm, grid_hbm, d_hbm,
                   sv0, sv1, idx0, idx1, pen0, pen1, dv0, dv1,
                   s0, s1, o0, o1, sp):
        wid = lax.axis_index("s") * _NC + lax.axis_index("c")
        base0 = wid * _NP
        svs = (sv0, sv1)
        idxs, pens, dvs = (idx0, idx1), (pen0, pen1), (dv0, dv1)
        sems, osems = (s0, s1), (o0, o1)

        row0 = wid * (_NP // C)

        def slab_src(k):
            return xt_hbm.at[:, pl.ds(row0 + k * _CROWS, _CROWS), :]

        pltpu.sync_copy(slab_src(0), sv0)
        pf = pltpu.async_copy(slab_src(1), sv1, sp)
        _sc_compute_chunk(sv0, idx0, pen0)
        cps = [pltpu.async_copy(grid_hbm.at[idx0], dv0, s0)]
        wrs = [None] * _NCH
        for k in range(1, _NCH):
            b = k % 2
            pf.wait()
            if k + 1 < _NCH:
                pf = pltpu.async_copy(slab_src(k + 1), svs[1 - b], sp)
            _sc_compute_chunk(svs[b], idxs[b], pens[b])
            if k >= 2:
                wrs[k - 2].wait()
            cps.append(pltpu.async_copy(grid_hbm.at[idxs[b]], dvs[b], sems[b]))
            cps[k - 1].wait()
            _sc_pass2(dvs[1 - b], pens[1 - b])
            wrs[k - 1] = pltpu.async_copy(
                dvs[1 - b], d_hbm.at[pl.ds(base0 + (k - 1) * _CH, _CH)],
                osems[1 - b])
        last = _NCH - 1
        cps[last].wait()
        _sc_pass2(dvs[last % 2], pens[last % 2])
        wrs[last - 1].wait()
        wrs[last] = pltpu.async_copy(
            dvs[last % 2], d_hbm.at[pl.ds(base0 + last * _CH, _CH)],
            osems[last % 2])
        wrs[last].wait()

    return _sc_gather


def kernel(xyz_ndc, grid):
    xt = jnp.transpose(xyz_ndc).reshape(3, R, C)
    d = _make_sc_gather()(xt, grid.reshape(N))
    thr = _tcb()
    out8 = _tcc(d.reshape(R, C), thr)
    return out8.reshape(N).view(jnp.bool_)
